# baseline (speedup 1.0000x reference)
import functools

import jax
import jax.numpy as jnp
from jax.experimental import pallas as pl
from jax.experimental.pallas import tpu as pltpu


def _tile_index(t, n_sweep1, C):
    in1 = t < n_sweep1
    r1 = t // C
    c1 = t % C
    tp = t - n_sweep1
    k = tp // (C + 1)
    j = tp % (C + 1)
    r2 = jnp.where(j <= k, j, j - k - 1)
    c2 = jnp.where(j <= k, k, C - 1 - k)
    return in1, jnp.where(in1, r1, r2), jnp.where(in1, c1, c2)


def _gcn_body(x_ref, adj_ref, w1_ref, b1_ref, w2_ref, b2_ref,
              out_ref, s1_ref, s2_ref, hacc_ref, *, block, C, n_sweep1):
    t = pl.program_id(0)
    in1, r, c = _tile_index(t, n_sweep1, C)
    n, nout = out_ref.shape

    @pl.when(t == 0)
    def _():
        s1_ref[...] = jnp.dot(x_ref[...], w1_ref[...],
                              preferred_element_type=jnp.float32)
        out_ref[...] = jnp.broadcast_to(b2_ref[...], (n, nout))

    tile = adj_ref[0, :, 0, 0, :]

    @pl.when(in1)
    def _():
        contrib = jnp.dot(tile, s1_ref[pl.ds(c * block, block), :],
                          preferred_element_type=jnp.float32)
        h_new = jnp.where(c == 0, contrib, hacc_ref[...] + contrib)
        hacc_ref[...] = h_new

        @pl.when(c == C - 1)
        def _():
            hrow = jnp.maximum(h_new + b1_ref[...], 0.0)
            s2_ref[pl.ds(r * block, block), :] = jnp.dot(
                hrow, w2_ref[...], preferred_element_type=jnp.float32)

    @pl.when(jnp.logical_or(jnp.logical_and(in1, c < r),
                            jnp.logical_not(in1)))
    def _():
        out_ref[pl.ds(r * block, block), :] += jnp.dot(
            tile, s2_ref[pl.ds(c * block, block), :],
            preferred_element_type=jnp.float32)


def kernel(x, adj, W1, b1, W2, b2):
    n, nfeat = x.shape
    hid1 = W1.shape[1]
    nout = W2.shape[1]

    block = next(b for b in (1000, 200, 100, 50, 10, 8, 2, 1)
                 if n % b == 0 and (n // b) % 2 == 0)
    C = n // block
    n_sweep1 = C * C
    n_steps = n_sweep1 + (C // 2) * (C + 1)

    def adj_index(t):
        _, r, c = _tile_index(t, n_sweep1, C)
        return (r, 0, c, 0, 0)

    adj5 = adj.reshape(C, block, C, 1, block)

    b1r = b1.reshape(1, hid1)
    b2r = b2.reshape(1, nout)

    out = pl.pallas_call(
        functools.partial(_gcn_body, block=block, C=C, n_sweep1=n_sweep1),
        grid=(n_steps,),
        in_specs=[
            pl.BlockSpec((n, nfeat), lambda t: (0, 0)),
            pl.BlockSpec((1, block, 1, 1, block), adj_index),
            pl.BlockSpec((nfeat, hid1), lambda t: (0, 0)),
            pl.BlockSpec((1, hid1), lambda t: (0, 0)),
            pl.BlockSpec((hid1, nout), lambda t: (0, 0)),
            pl.BlockSpec((1, nout), lambda t: (0, 0)),
        ],
        out_specs=pl.BlockSpec((n, nout), lambda t: (0, 0)),
        out_shape=jax.ShapeDtypeStruct((n, nout), jnp.float32),
        scratch_shapes=[
            pltpu.VMEM((n, hid1), jnp.float32),
            pltpu.VMEM((n, nout), jnp.float32),
            pltpu.VMEM((block, hid1), jnp.float32),
        ],
        compiler_params=pltpu.CompilerParams(
            dimension_semantics=("arbitrary",),
        ),
    )(x, adj5, W1, b1r, W2, b2r)
    return out


# speedup vs baseline: 4.3841x; 4.3841x over previous
import functools

import jax
import jax.numpy as jnp
from jax.experimental import pallas as pl
from jax.experimental.pallas import tpu as pltpu


def _sweep1_body(x_ref, adj_ref, w1_ref, b1_ref, w2_ref, b2_ref,
                 emb_ref, s2_ref, s1_ref, *, block):
    r = pl.program_id(0)
    n, nout = emb_ref.shape

    @pl.when(r == 0)
    def _():
        s1_ref[...] = jnp.dot(x_ref[...], w1_ref[...],
                              preferred_element_type=jnp.float32)
        s2_ref[...] = jnp.zeros_like(s2_ref)

    adj_blk = adj_ref[...]
    emb_ref[pl.ds(r * block, block), :] = (
        jnp.dot(adj_blk, s2_ref[...], preferred_element_type=jnp.float32)
        + b2_ref[...])
    h = jnp.maximum(
        jnp.dot(adj_blk, s1_ref[...], preferred_element_type=jnp.float32)
        + b1_ref[...], 0.0)
    s2_ref[pl.ds(r * block, block), :] = jnp.dot(
        h, w2_ref[...], preferred_element_type=jnp.float32)


def _sweep2_body(adj_ref, s2p_ref, embp_ref, out_ref, *,
                 block, cw, cum, cmin):
    t = pl.program_id(0)
    r = sum(jnp.where(t >= s, 1, 0) for s in cum[1:])
    cum_r = sum(jnp.where(r == i, s, 0) for i, s in enumerate(cum))
    cmin_r = sum(jnp.where(r == i, s, 0) for i, s in enumerate(cmin))
    c = cmin_r + (t - cum_r)

    @pl.when(t == 0)
    def _():
        out_ref[...] = embp_ref[...]

    n = out_ref.shape[0]
    nctiles = s2p_ref.shape[0] // cw
    s2_slice = s2p_ref[pl.ds(c * cw, cw), :]
    row_ids = c * cw + jax.lax.broadcasted_iota(jnp.int32, (cw, 1), 0)
    s2m = jnp.where(row_ids >= r * block, s2_slice, 0.0)

    @pl.when(c < nctiles - 1)
    def _():
        out_ref[pl.ds(r * block, block), :] += jnp.dot(
            adj_ref[...], s2m, preferred_element_type=jnp.float32)

    @pl.when(c == nctiles - 1)
    def _():
        col_ids = c * cw + jax.lax.broadcasted_iota(
            jnp.int32, adj_ref.shape, 1)
        tile = jnp.where(col_ids < n, adj_ref[...], 0.0)
        out_ref[pl.ds(r * block, block), :] += jnp.dot(
            tile, s2m, preferred_element_type=jnp.float32)


def _sweep2_index(t, cum, cmin):
    r = sum(jnp.where(t >= s, 1, 0) for s in cum[1:])
    cum_r = sum(jnp.where(r == i, s, 0) for i, s in enumerate(cum))
    cmin_r = sum(jnp.where(r == i, s, 0) for i, s in enumerate(cmin))
    return (r, cmin_r + (t - cum_r))


def kernel(x, adj, W1, b1, W2, b2):
    n, nfeat = x.shape
    hid1 = W1.shape[1]
    nout = W2.shape[1]

    block = next(b for b in (400, 100, 40, 8, 1) if n % b == 0)
    nblocks = n // block
    cw = 512 if n >= 512 else 64 if n >= 64 else 8
    nctiles = -(-n // cw)
    cmin = tuple((block * r) // cw for r in range(nblocks))
    counts = [nctiles - cm for cm in cmin]
    cum = []
    acc = 0
    for cnt in counts:
        cum.append(acc)
        acc += cnt
    cum = tuple(cum)
    n_sweep2 = acc

    b1r = b1.reshape(1, hid1)
    b2r = b2.reshape(1, nout)

    emb_part, s2 = pl.pallas_call(
        functools.partial(_sweep1_body, block=block),
        grid=(nblocks,),
        in_specs=[
            pl.BlockSpec((n, nfeat), lambda r: (0, 0)),
            pl.BlockSpec((block, n), lambda r: (r, 0)),
            pl.BlockSpec((nfeat, hid1), lambda r: (0, 0)),
            pl.BlockSpec((1, hid1), lambda r: (0, 0)),
            pl.BlockSpec((hid1, nout), lambda r: (0, 0)),
            pl.BlockSpec((1, nout), lambda r: (0, 0)),
        ],
        out_specs=[
            pl.BlockSpec((n, nout), lambda r: (0, 0)),
            pl.BlockSpec((n, nout), lambda r: (0, 0)),
        ],
        out_shape=[
            jax.ShapeDtypeStruct((n, nout), jnp.float32),
            jax.ShapeDtypeStruct((n, nout), jnp.float32),
        ],
        scratch_shapes=[pltpu.VMEM((n, hid1), jnp.float32)],
        compiler_params=pltpu.CompilerParams(
            dimension_semantics=("arbitrary",),
        ),
    )(x, adj, W1, b1r, W2, b2r)

    s2p = jnp.pad(s2, ((0, nctiles * cw - n), (0, 0)))

    out = pl.pallas_call(
        functools.partial(_sweep2_body, block=block, cw=cw,
                          cum=cum, cmin=cmin),
        grid=(n_sweep2,),
        in_specs=[
            pl.BlockSpec((block, cw),
                         functools.partial(_sweep2_index, cum=cum,
                                           cmin=cmin)),
            pl.BlockSpec((nctiles * cw, nout), lambda t: (0, 0)),
            pl.BlockSpec((n, nout), lambda t: (0, 0)),
        ],
        out_specs=pl.BlockSpec((n, nout), lambda t: (0, 0)),
        out_shape=jax.ShapeDtypeStruct((n, nout), jnp.float32),
        compiler_params=pltpu.CompilerParams(
            dimension_semantics=("arbitrary",),
        ),
    )(adj, s2p, emb_part)
    return out


# speedup vs baseline: 6.2860x; 1.4338x over previous
import functools

import jax
import jax.numpy as jnp
from jax.experimental import pallas as pl
from jax.experimental.pallas import tpu as pltpu


def _sweep1_body(x_ref, adj_ref, w1_ref, b1_ref, w2_ref, b2_ref,
                 emb_ref, s2_ref, s1_ref, *, block):
    r = pl.program_id(0)
    n, nout = emb_ref.shape

    @pl.when(r == 0)
    def _():
        s1_ref[...] = jnp.dot(x_ref[...], w1_ref[...],
                              preferred_element_type=jnp.float32)
        s2_ref[...] = jnp.zeros_like(s2_ref)

    adj_blk = adj_ref[...]
    emb_ref[pl.ds(r * block, block), :] = (
        jnp.dot(adj_blk, s2_ref[...], preferred_element_type=jnp.float32)
        + b2_ref[...])
    h = jnp.maximum(
        jnp.dot(adj_blk, s1_ref[...], preferred_element_type=jnp.float32)
        + b1_ref[...], 0.0)
    s2_ref[pl.ds(r * block, block), :] = jnp.dot(
        h, w2_ref[...], preferred_element_type=jnp.float32)


def _sweep2_body(adj_ref, s2p_ref, embp_ref, out_ref, *,
                 block, cw, cum, cmin):
    t = pl.program_id(0)
    r = sum(jnp.where(t >= s, 1, 0) for s in cum[1:])
    cum_r = sum(jnp.where(r == i, s, 0) for i, s in enumerate(cum))
    cmin_r = sum(jnp.where(r == i, s, 0) for i, s in enumerate(cmin))
    c = cmin_r + (t - cum_r)

    @pl.when(t == 0)
    def _():
        out_ref[...] = embp_ref[...]

    n = out_ref.shape[0]
    nctiles = s2p_ref.shape[0] // cw
    s2_slice = s2p_ref[pl.ds(c * cw, cw), :]
    row_ids = c * cw + jax.lax.broadcasted_iota(jnp.int32, (cw, 1), 0)
    s2m = jnp.where(row_ids >= r * block, s2_slice, 0.0)

    @pl.when(c < nctiles - 1)
    def _():
        out_ref[pl.ds(r * block, block), :] += jnp.dot(
            adj_ref[...], s2m, preferred_element_type=jnp.float32)

    @pl.when(c == nctiles - 1)
    def _():
        col_ids = c * cw + jax.lax.broadcasted_iota(
            jnp.int32, adj_ref.shape, 1)
        tile = jnp.where(col_ids < n, adj_ref[...], 0.0)
        out_ref[pl.ds(r * block, block), :] += jnp.dot(
            tile, s2m, preferred_element_type=jnp.float32)


def _sweep2_index(t, cum, cmin):
    r = sum(jnp.where(t >= s, 1, 0) for s in cum[1:])
    cum_r = sum(jnp.where(r == i, s, 0) for i, s in enumerate(cum))
    cmin_r = sum(jnp.where(r == i, s, 0) for i, s in enumerate(cmin))
    return (r, cmin_r + (t - cum_r))


def kernel(x, adj, W1, b1, W2, b2):
    n, nfeat = x.shape
    hid1 = W1.shape[1]
    nout = W2.shape[1]

    block = next(b for b in (400, 100, 40, 8, 1) if n % b == 0)
    nblocks = n // block
    cw = 2048 if n >= 2048 else 64 if n >= 64 else 8
    nctiles = -(-n // cw)
    cmin = tuple((block * r) // cw for r in range(nblocks))
    counts = [nctiles - cm for cm in cmin]
    cum = []
    acc = 0
    for cnt in counts:
        cum.append(acc)
        acc += cnt
    cum = tuple(cum)
    n_sweep2 = acc

    b1r = b1.reshape(1, hid1)
    b2r = b2.reshape(1, nout)

    emb_part, s2 = pl.pallas_call(
        functools.partial(_sweep1_body, block=block),
        grid=(nblocks,),
        in_specs=[
            pl.BlockSpec((n, nfeat), lambda r: (0, 0)),
            pl.BlockSpec((block, n), lambda r: (r, 0)),
            pl.BlockSpec((nfeat, hid1), lambda r: (0, 0)),
            pl.BlockSpec((1, hid1), lambda r: (0, 0)),
            pl.BlockSpec((hid1, nout), lambda r: (0, 0)),
            pl.BlockSpec((1, nout), lambda r: (0, 0)),
        ],
        out_specs=[
            pl.BlockSpec((n, nout), lambda r: (0, 0)),
            pl.BlockSpec((n, nout), lambda r: (0, 0)),
        ],
        out_shape=[
            jax.ShapeDtypeStruct((n, nout), jnp.float32),
            jax.ShapeDtypeStruct((n, nout), jnp.float32),
        ],
        scratch_shapes=[pltpu.VMEM((n, hid1), jnp.float32)],
        compiler_params=pltpu.CompilerParams(
            dimension_semantics=("arbitrary",),
        ),
    )(x, adj, W1, b1r, W2, b2r)

    s2p = jnp.pad(s2, ((0, nctiles * cw - n), (0, 0)))

    out = pl.pallas_call(
        functools.partial(_sweep2_body, block=block, cw=cw,
                          cum=cum, cmin=cmin),
        grid=(n_sweep2,),
        in_specs=[
            pl.BlockSpec((block, cw),
                         functools.partial(_sweep2_index, cum=cum,
                                           cmin=cmin)),
            pl.BlockSpec((nctiles * cw, nout), lambda t: (0, 0)),
            pl.BlockSpec((n, nout), lambda t: (0, 0)),
        ],
        out_specs=pl.BlockSpec((n, nout), lambda t: (0, 0)),
        out_shape=jax.ShapeDtypeStruct((n, nout), jnp.float32),
        compiler_params=pltpu.CompilerParams(
            dimension_semantics=("arbitrary",),
        ),
    )(adj, s2p, emb_part)
    return out
